# baseline (speedup 1.0000x reference)
import jax
import jax.numpy as jnp
from jax import lax
from jax.experimental import pallas as pl


def _tc_chamfer_body(a_ref, b_ref, d1_ref, d2_ref):
    i = pl.program_id(1)
    a = -2.0 * a_ref[0]
    b = b_ref[0]
    asq = 0.25 * jnp.sum(a * a, axis=1)
    bsq = jnp.sum(b * b, axis=1)
    cross = lax.dot_general(a, b, (((1,), (1,)), ((), ())))
    d = (cross + asq[:, None]) + bsq[None, :]
    d1_ref[0, 0, :] = jnp.maximum(jnp.min(d, axis=1), 0.0)
    colpart = jnp.maximum(jnp.min(d, axis=0), 0.0)

    @pl.when(i == 0)
    def _():
        d2_ref[0, 0, :] = colpart

    @pl.when(i > 0)
    def _():
        d2_ref[0, 0, :] = jnp.minimum(d2_ref[0, 0, :], colpart)


def kernel(xyz1, xyz2):
    B, N, _ = xyz1.shape
    M = xyz2.shape[1]
    IB = 512
    ni = N // IB
    d1, d2 = pl.pallas_call(
        _tc_chamfer_body,
        grid=(B, ni),
        in_specs=[
            pl.BlockSpec((1, IB, 3), lambda b, i: (b, i, 0)),
            pl.BlockSpec((1, M, 3), lambda b, i: (b, 0, 0)),
        ],
        out_specs=[
            pl.BlockSpec((1, 1, IB), lambda b, i: (b * ni + i, 0, 0)),
            pl.BlockSpec((1, 1, M), lambda b, i: (b, 0, 0)),
        ],
        out_shape=[
            jax.ShapeDtypeStruct((B * ni, 1, IB), jnp.float32),
            jax.ShapeDtypeStruct((B, 1, M), jnp.float32),
        ],
    )(xyz1, xyz2)
    return d1.reshape(B, N), d2.reshape(B, M)


# speedup vs baseline: 1.0881x; 1.0881x over previous
import jax
import jax.numpy as jnp
from jax import lax
from jax.experimental import pallas as pl


def _tc_chamfer_body(a_ref, bt_ref, d1_ref, d2_ref):
    i = pl.program_id(1)
    a = a_ref[0]
    bt = bt_ref[0]
    asq = 0.25 * jnp.sum(a * a, axis=1)
    bsq = jnp.sum(bt * bt, axis=0)
    cross = lax.dot_general(a, bt, (((1,), (0,)), ((), ())))
    d = (cross + asq[:, None]) + bsq[None, :]
    d1_ref[0, 0, :] = jnp.maximum(jnp.min(d, axis=1), 0.0)
    colpart = jnp.maximum(jnp.min(d, axis=0), 0.0)

    @pl.when(i == 0)
    def _():
        d2_ref[0, 0, :] = colpart

    @pl.when(i > 0)
    def _():
        d2_ref[0, 0, :] = jnp.minimum(d2_ref[0, 0, :], colpart)


def kernel(xyz1, xyz2):
    B, N, _ = xyz1.shape
    M = xyz2.shape[1]
    IB = 1024
    ni = N // IB
    a2 = -2.0 * xyz1
    bt = jnp.transpose(xyz2, (0, 2, 1))
    d1, d2 = pl.pallas_call(
        _tc_chamfer_body,
        grid=(B, ni),
        in_specs=[
            pl.BlockSpec((1, IB, 3), lambda b, i: (b, i, 0)),
            pl.BlockSpec((1, 3, M), lambda b, i: (b, 0, 0)),
        ],
        out_specs=[
            pl.BlockSpec((1, 1, IB), lambda b, i: (b * ni + i, 0, 0)),
            pl.BlockSpec((1, 1, M), lambda b, i: (b, 0, 0)),
        ],
        out_shape=[
            jax.ShapeDtypeStruct((B * ni, 1, IB), jnp.float32),
            jax.ShapeDtypeStruct((B, 1, M), jnp.float32),
        ],
    )(a2, bt)
    return d1.reshape(B, N), d2.reshape(B, M)


# speedup vs baseline: 1.1414x; 1.0489x over previous
import jax
import jax.numpy as jnp
from jax import lax
from jax.experimental import pallas as pl
from jax.experimental.pallas import tpu as pltpu


def _tc_chamfer_body(a_ref, b_ref, d1_ref, d2_ref, bt_ref):
    i = pl.program_id(1)

    @pl.when(i == 0)
    def _():
        b = b_ref[0]
        bt = jnp.transpose(b)
        bt_ref[0:3, :] = bt
        bt_ref[3:4, :] = jnp.sum(bt * bt, axis=0, keepdims=True)

    bt = bt_ref[0:3, :]
    bsq = bt_ref[3, :]
    a = -2.0 * a_ref[0]
    asq = 0.25 * jnp.sum(a * a, axis=1)
    cross = lax.dot_general(a, bt, (((1,), (0,)), ((), ())))
    d = (cross + asq[:, None]) + bsq[None, :]
    d1_ref[0, 0, :] = jnp.maximum(jnp.min(d, axis=1), 0.0)
    colpart = jnp.maximum(jnp.min(d, axis=0), 0.0)

    @pl.when(i == 0)
    def _():
        d2_ref[0, 0, :] = colpart

    @pl.when(i > 0)
    def _():
        d2_ref[0, 0, :] = jnp.minimum(d2_ref[0, 0, :], colpart)


def kernel(xyz1, xyz2):
    B, N, _ = xyz1.shape
    M = xyz2.shape[1]
    IB = 512
    ni = N // IB
    d1, d2 = pl.pallas_call(
        _tc_chamfer_body,
        grid=(B, ni),
        in_specs=[
            pl.BlockSpec((1, IB, 3), lambda b, i: (b, i, 0)),
            pl.BlockSpec((1, M, 3), lambda b, i: (b, 0, 0)),
        ],
        out_specs=[
            pl.BlockSpec((1, 1, IB), lambda b, i: (b * ni + i, 0, 0)),
            pl.BlockSpec((1, 1, M), lambda b, i: (b, 0, 0)),
        ],
        out_shape=[
            jax.ShapeDtypeStruct((B * ni, 1, IB), jnp.float32),
            jax.ShapeDtypeStruct((B, 1, M), jnp.float32),
        ],
        scratch_shapes=[pltpu.VMEM((8, M), jnp.float32)],
    )(xyz1, xyz2)
    return d1.reshape(B, N), d2.reshape(B, M)


# speedup vs baseline: 1.1996x; 1.0510x over previous
import jax
import jax.numpy as jnp
from jax import lax
from jax.experimental import pallas as pl
from jax.experimental.pallas import tpu as pltpu


def _tc_chamfer_body(a_ref, b_ref, d1_ref, d2_ref, bt_ref):
    i = pl.program_id(1)

    @pl.when(i == 0)
    def _():
        b = b_ref[0]
        bt = jnp.transpose(b)
        bt_ref[0:3, :] = bt
        bt_ref[3:4, :] = jnp.sum(bt * bt, axis=0, keepdims=True)

    b_id = pl.program_id(0)
    IB = a_ref.shape[1]
    bt = bt_ref[0:3, :]
    bsq = bt_ref[3, :]
    a = -2.0 * a_ref[0]
    asq = 0.25 * jnp.sum(a * a, axis=1)
    cross = lax.dot_general(a, bt, (((1,), (0,)), ((), ())))
    d = (cross + asq[:, None]) + bsq[None, :]
    d1_ref[b_id, pl.ds(i * IB, IB)] = jnp.maximum(jnp.min(d, axis=1), 0.0)
    colpart = jnp.maximum(jnp.min(d, axis=0), 0.0)

    @pl.when(i == 0)
    def _():
        d2_ref[b_id, :] = colpart

    @pl.when(i > 0)
    def _():
        d2_ref[b_id, :] = jnp.minimum(d2_ref[b_id, :], colpart)


def kernel(xyz1, xyz2):
    B, N, _ = xyz1.shape
    M = xyz2.shape[1]
    IB = 512
    ni = N // IB
    d1, d2 = pl.pallas_call(
        _tc_chamfer_body,
        grid=(B, ni),
        in_specs=[
            pl.BlockSpec((1, IB, 3), lambda b, i: (b, i, 0)),
            pl.BlockSpec((1, M, 3), lambda b, i: (b, 0, 0)),
        ],
        out_specs=[
            pl.BlockSpec((B, N), lambda b, i: (0, 0)),
            pl.BlockSpec((B, M), lambda b, i: (0, 0)),
        ],
        out_shape=[
            jax.ShapeDtypeStruct((B, N), jnp.float32),
            jax.ShapeDtypeStruct((B, M), jnp.float32),
        ],
        scratch_shapes=[pltpu.VMEM((8, M), jnp.float32)],
    )(xyz1, xyz2)
    return d1, d2


# speedup vs baseline: 1.2067x; 1.0059x over previous
import jax
import jax.numpy as jnp
from jax import lax
from jax.experimental import pallas as pl
from jax.experimental.pallas import tpu as pltpu


def _tc_chamfer_body(a_ref, b_ref, d1_ref, d2_ref, bt_ref, btb_ref):
    i = pl.program_id(1)

    @pl.when(i == 0)
    def _():
        b = b_ref[0]
        bt = jnp.transpose(b)
        btb_ref[0:3, :] = bt.astype(jnp.bfloat16)
        bt_ref[0:1, :] = jnp.sum(bt * bt, axis=0, keepdims=True)

    b_id = pl.program_id(0)
    IB = a_ref.shape[1]
    btb = btb_ref[0:3, :]
    bsq = bt_ref[0, :]
    a = -2.0 * a_ref[0]
    asq = 0.25 * jnp.sum(a * a, axis=1)
    ab = a.astype(jnp.bfloat16)
    cross = lax.dot_general(ab, btb, (((1,), (0,)), ((), ())),
                            preferred_element_type=jnp.float32)
    d = (cross + asq[:, None]) + bsq[None, :]
    d1_ref[b_id, pl.ds(i * IB, IB)] = jnp.maximum(jnp.min(d, axis=1), 0.0)
    colpart = jnp.maximum(jnp.min(d, axis=0), 0.0)

    @pl.when(i == 0)
    def _():
        d2_ref[b_id, :] = colpart

    @pl.when(i > 0)
    def _():
        d2_ref[b_id, :] = jnp.minimum(d2_ref[b_id, :], colpart)


def kernel(xyz1, xyz2):
    B, N, _ = xyz1.shape
    M = xyz2.shape[1]
    IB = 512
    ni = N // IB
    d1, d2 = pl.pallas_call(
        _tc_chamfer_body,
        grid=(B, ni),
        in_specs=[
            pl.BlockSpec((1, IB, 3), lambda b, i: (b, i, 0)),
            pl.BlockSpec((1, M, 3), lambda b, i: (b, 0, 0)),
        ],
        out_specs=[
            pl.BlockSpec((B, N), lambda b, i: (0, 0)),
            pl.BlockSpec((B, M), lambda b, i: (0, 0)),
        ],
        out_shape=[
            jax.ShapeDtypeStruct((B, N), jnp.float32),
            jax.ShapeDtypeStruct((B, M), jnp.float32),
        ],
        scratch_shapes=[pltpu.VMEM((8, M), jnp.float32),
                        pltpu.VMEM((16, M), jnp.bfloat16)],
    )(xyz1, xyz2)
    return d1, d2


# speedup vs baseline: 1.2225x; 1.0131x over previous
import jax
import jax.numpy as jnp
from jax import lax
from jax.experimental import pallas as pl
from jax.experimental.pallas import tpu as pltpu


def _hi_lo(x):
    hi = x.astype(jnp.bfloat16)
    lo = (x - hi.astype(jnp.float32)).astype(jnp.bfloat16)
    return hi, lo


def _tc_chamfer_body(a_ref, b_ref, d1_ref, d2_ref, btb_ref):
    b_id = pl.program_id(0)
    i = pl.program_id(1)
    IB = a_ref.shape[1]
    M = b_ref.shape[1]

    @pl.when(i == 0)
    def _():
        b = b_ref[0]
        bt = jnp.transpose(b)
        bsq = jnp.sum(bt * bt, axis=0, keepdims=True)
        bh, bl = _hi_lo(bsq)
        one = jnp.ones((2, M), dtype=jnp.bfloat16)
        zero = jnp.zeros((1, M), dtype=jnp.bfloat16)
        btb_ref[...] = jnp.concatenate(
            [bt.astype(jnp.bfloat16), one, bh, bl, zero], axis=0)

    a = -2.0 * a_ref[0]
    asq = 0.25 * jnp.sum(a * a, axis=1, keepdims=True)
    ah, al = _hi_lo(asq)
    aone = jnp.ones((IB, 2), dtype=jnp.bfloat16)
    azero = jnp.zeros((IB, 1), dtype=jnp.bfloat16)
    amat = jnp.concatenate([a.astype(jnp.bfloat16), ah, al, aone, azero],
                           axis=1)
    d = lax.dot_general(amat, btb_ref[...], (((1,), (0,)), ((), ())),
                        preferred_element_type=jnp.float32)
    d1_ref[b_id, pl.ds(i * IB, IB)] = jnp.maximum(jnp.min(d, axis=1), 0.0)
    colpart = jnp.maximum(jnp.min(d, axis=0), 0.0)

    @pl.when(i == 0)
    def _():
        d2_ref[b_id, :] = colpart

    @pl.when(i > 0)
    def _():
        d2_ref[b_id, :] = jnp.minimum(d2_ref[b_id, :], colpart)


def kernel(xyz1, xyz2):
    B, N, _ = xyz1.shape
    M = xyz2.shape[1]
    IB = 512
    ni = N // IB
    d1, d2 = pl.pallas_call(
        _tc_chamfer_body,
        grid=(B, ni),
        in_specs=[
            pl.BlockSpec((1, IB, 3), lambda b, i: (b, i, 0)),
            pl.BlockSpec((1, M, 3), lambda b, i: (b, 0, 0)),
        ],
        out_specs=[
            pl.BlockSpec((B, N), lambda b, i: (0, 0)),
            pl.BlockSpec((B, M), lambda b, i: (0, 0)),
        ],
        out_shape=[
            jax.ShapeDtypeStruct((B, N), jnp.float32),
            jax.ShapeDtypeStruct((B, M), jnp.float32),
        ],
        scratch_shapes=[pltpu.VMEM((8, M), jnp.bfloat16)],
    )(xyz1, xyz2)
    return d1, d2
